# Initial kernel scaffold; baseline (speedup 1.0000x reference)
import jax
import jax.numpy as jnp
from jax.experimental import pallas as pl


def kernel(x_t, log_weights, obs, noise, u0, A):
    raise NotImplementedError("write your pallas kernel here")


# speedup vs baseline: 38.8322x; 38.8322x over previous
import functools

import jax
import jax.numpy as jnp
from jax import lax
from jax.experimental import pallas as pl
from jax.experimental.pallas import tpu as pltpu
from jax.experimental.pallas import tpu_sc as plsc

_B, _N, _D = 32, 65536, 16
_RPW = 512
_CH_S = 8192
_G_ROWS = 2048
_H_PAD = _N + 128


def _s_body(cdf_ref, u0_ref, s_ref):
    v = cdf_ref[...] * 65536.0
    u0 = u0_ref[0, 0, 0]
    gf = jnp.floor(v - u0)
    gc = jnp.clip(gf, 0.0, 65535.0)
    acc = jnp.maximum(gc - 2.0, 0.0)
    for d in range(-2, 3):
        jp = gc + float(d)
        ok = (jp >= 0.0) & (jp <= 65535.0) & ((jp + u0) <= v)
        acc = acc + jnp.where(ok, 1.0, 0.0)
    s_ref[...] = acc.astype(jnp.int32)


def _compute_s(cdf_norm, u0):
    cdf_v = cdf_norm.reshape(_B, _RPW, 128)
    u0_v = u0.reshape(_B, 1, 1)
    return pl.pallas_call(
        _s_body,
        grid=(_B,),
        in_specs=[
            pl.BlockSpec((1, _RPW, 128), lambda b: (b, 0, 0)),
            pl.BlockSpec((1, 1, 1), lambda b: (b, 0, 0)),
        ],
        out_specs=pl.BlockSpec((1, _RPW, 128), lambda b: (b, 0, 0)),
        out_shape=jax.ShapeDtypeStruct((_B, _RPW, 128), jnp.int32),
    )(cdf_v, u0_v)


def _sc_resample(s_hbm, x_hbm, out_hbm, h, sbuf, gbuf, sem):
    wid = lax.axis_index("s") * 2 + lax.axis_index("c")
    base = wid * _N
    zeros16 = jnp.zeros((16,), jnp.int32)
    ones16 = jnp.ones((16,), jnp.int32)

    def zero_body(i, c):
        h[pl.ds(i * 16, 16)] = zeros16
        return c

    lax.fori_loop(0, _H_PAD // 16, zero_body, 0)

    def chunk_body(c, carry):
        pltpu.sync_copy(s_hbm.at[pl.ds(base + c * _CH_S, _CH_S)], sbuf)

        def scat(i, cc):
            sv = sbuf[pl.ds(i * 16, 16)]
            plsc.addupdate_scatter(h, [sv], ones16)
            return cc

        lax.fori_loop(0, _CH_S // 16, scat, 0)
        return carry

    lax.fori_loop(0, _N // _CH_S, chunk_body, 0)

    def scan_body(i, carry):
        v = h[pl.ds(i * 16, 16)]
        cs = plsc.cumsum(v)
        h[pl.ds(i * 16, 16)] = cs + (carry + base)
        return carry + lax.reduce_max(cs, (0,))

    lax.fori_loop(0, _N // 16, scan_body, 0)

    def gout(t, carry):
        def fire(m, cc):
            idxs = h.at[pl.ds(t * _G_ROWS + m * 128, 128)]
            cp = pltpu.async_copy(
                x_hbm.at[idxs], gbuf.at[pl.ds(m * 128, 128), :], sem)
            cp.wait()
            return cc

        lax.fori_loop(0, _G_ROWS // 128, fire, 0)
        pltpu.sync_copy(
            gbuf, out_hbm.at[pl.ds(base + t * _G_ROWS, _G_ROWS), :])
        return carry

    lax.fori_loop(0, _N // _G_ROWS, gout, 0)


def _resample_gather(s, x_flat):
    mesh = plsc.VectorSubcoreMesh(core_axis_name="c", subcore_axis_name="s")
    kern = functools.partial(
        pl.kernel,
        mesh=mesh,
        out_type=jax.ShapeDtypeStruct((_B * _N, _D), jnp.float32),
        scratch_types=[
            pltpu.VMEM((_H_PAD,), jnp.int32),
            pltpu.VMEM((_CH_S,), jnp.int32),
            pltpu.VMEM((_G_ROWS, _D), jnp.float32),
            pltpu.SemaphoreType.DMA,
        ],
        compiler_params=pltpu.CompilerParams(
            needs_layout_passes=False, use_tc_tiling_on_sc=False),
    )(_sc_resample)
    return kern(s.reshape(_B * _N), x_flat)


def _prop_body(xr_ref, nz_ref, obs_ref, ab_ref, t_ref, xn_ref, lnw_ref):
    x = xr_ref[0]
    z = jnp.dot(x, ab_ref[...], preferred_element_type=jnp.float32)
    xn = z + 0.1 * nz_ref[0]
    xn_ref[0] = xn
    dfv = xn - obs_ref[0]
    sq = dfv * dfv
    q = jnp.dot(sq, t_ref[...], preferred_element_type=jnp.float32)
    g = -0.5 * q
    m = jnp.max(g)
    lse = jnp.log(jnp.sum(jnp.exp(g - m))) + m
    lnw_ref[0] = g - lse


def _propagate(xr_v, nz_v, obs_t, a_big, t_sel):
    return pl.pallas_call(
        _prop_body,
        grid=(_B,),
        in_specs=[
            pl.BlockSpec((1, _N // 8, 128), lambda b: (b, 0, 0)),
            pl.BlockSpec((1, _N // 8, 128), lambda b: (b, 0, 0)),
            pl.BlockSpec((1, 1, 128), lambda b: (b, 0, 0)),
            pl.BlockSpec((128, 128), lambda b: (0, 0)),
            pl.BlockSpec((128, 8), lambda b: (0, 0)),
        ],
        out_specs=[
            pl.BlockSpec((1, _N // 8, 128), lambda b: (b, 0, 0)),
            pl.BlockSpec((1, _N // 8, 8), lambda b: (b, 0, 0)),
        ],
        out_shape=[
            jax.ShapeDtypeStruct((_B, _N // 8, 128), jnp.float32),
            jax.ShapeDtypeStruct((_B, _N // 8, 8), jnp.float32),
        ],
    )(xr_v, nz_v, obs_t, a_big, t_sel)


def kernel(x_t, log_weights, obs, noise, u0, A):
    lnw = log_weights - jax.scipy.special.logsumexp(
        log_weights, axis=-1, keepdims=True)
    w = jnp.exp(lnw)
    cdf = jnp.cumsum(w, axis=1)
    cdf_norm = cdf / cdf[:, -1:]

    s = _compute_s(cdf_norm, u0)

    x_res = _resample_gather(s, x_t.reshape(_B * _N, _D))

    a_big = jnp.kron(jnp.eye(8, dtype=jnp.float32), A)
    t_sel = jnp.kron(jnp.eye(8, dtype=jnp.float32),
                     jnp.ones((16, 1), jnp.float32))
    obs_t = jnp.tile(obs, (1, 8)).reshape(_B, 1, 128)
    xn, lnw_new = _propagate(
        x_res.reshape(_B, _N // 8, 128),
        noise.reshape(_B, _N // 8, 128),
        obs_t, a_big, t_sel)

    return jnp.concatenate(
        [xn.reshape(_B, _N, _D), lnw_new.reshape(_B, _N, 1)], axis=-1)


# speedup vs baseline: 41.9770x; 1.0810x over previous
import functools

import jax
import jax.numpy as jnp
from jax import lax
from jax.experimental import pallas as pl
from jax.experimental.pallas import tpu as pltpu
from jax.experimental.pallas import tpu_sc as plsc

_B, _N, _D = 32, 65536, 16
_RPW = 512
_CH_S = 8192
_G_ROWS = 2048
_H_PAD = _N + 128


def _s_body(cdf_ref, u0_ref, s_ref):
    v = cdf_ref[...] * 65536.0
    u0 = u0_ref[0, 0, 0]
    gf = jnp.floor(v - u0)
    gc = jnp.clip(gf, 0.0, 65535.0)
    acc = jnp.maximum(gc - 2.0, 0.0)
    for d in range(-2, 3):
        jp = gc + float(d)
        ok = (jp >= 0.0) & (jp <= 65535.0) & ((jp + u0) <= v)
        acc = acc + jnp.where(ok, 1.0, 0.0)
    s_ref[...] = acc.astype(jnp.int32)


def _compute_s(cdf_norm, u0):
    cdf_v = cdf_norm.reshape(_B, _RPW, 128)
    u0_v = u0.reshape(_B, 1, 1)
    return pl.pallas_call(
        _s_body,
        grid=(_B,),
        in_specs=[
            pl.BlockSpec((1, _RPW, 128), lambda b: (b, 0, 0)),
            pl.BlockSpec((1, 1, 1), lambda b: (b, 0, 0)),
        ],
        out_specs=pl.BlockSpec((1, _RPW, 128), lambda b: (b, 0, 0)),
        out_shape=jax.ShapeDtypeStruct((_B, _RPW, 128), jnp.int32),
    )(cdf_v, u0_v)


def _sc_resample(s_hbm, x_hbm, z_hbm, out_hbm, h, sbuf, gbuf, sem):
    wid = lax.axis_index("s") * 2 + lax.axis_index("c")
    base = wid * _N
    ones16 = jnp.ones((16,), jnp.int32)

    pltpu.sync_copy(z_hbm, h)

    def chunk_body(c, carry):
        pltpu.sync_copy(s_hbm.at[pl.ds(base + c * _CH_S, _CH_S)], sbuf)

        def scat(i, cc):
            sv = sbuf[pl.ds(i * 16, 16)]
            plsc.addupdate_scatter(h, [sv], ones16)
            return cc

        lax.fori_loop(0, _CH_S // 16, scat, 0)
        return carry

    lax.fori_loop(0, _N // _CH_S, chunk_body, 0)

    def scan_body(i, carry):
        v = h[pl.ds(i * 16, 16)]
        cs = plsc.cumsum(v)
        h[pl.ds(i * 16, 16)] = cs + (carry + base)
        return carry + lax.reduce_max(cs, (0,))

    lax.fori_loop(0, _N // 16, scan_body, 0)

    def gout(t, carry):
        cps = []
        for m in range(_G_ROWS // 128):
            idxs = h.at[pl.ds(t * _G_ROWS + m * 128, 128)]
            cps.append(pltpu.async_copy(
                x_hbm.at[idxs], gbuf.at[pl.ds(m * 128, 128), :], sem))
        for cp in cps:
            cp.wait()
        pltpu.sync_copy(
            gbuf, out_hbm.at[pl.ds(base + t * _G_ROWS, _G_ROWS), :])
        return carry

    lax.fori_loop(0, _N // _G_ROWS, gout, 0)


def _resample_gather(s, x_flat):
    mesh = plsc.VectorSubcoreMesh(core_axis_name="c", subcore_axis_name="s")
    kern = functools.partial(
        pl.kernel,
        mesh=mesh,
        out_type=jax.ShapeDtypeStruct((_B * _N, _D), jnp.float32),
        scratch_types=[
            pltpu.VMEM((_H_PAD,), jnp.int32),
            pltpu.VMEM((_CH_S,), jnp.int32),
            pltpu.VMEM((_G_ROWS, _D), jnp.float32),
            pltpu.SemaphoreType.DMA,
        ],
        compiler_params=pltpu.CompilerParams(
            needs_layout_passes=False, use_tc_tiling_on_sc=False),
    )(_sc_resample)
    zeros_h = jnp.zeros((_H_PAD,), jnp.int32)
    return kern(s.reshape(_B * _N), x_flat, zeros_h)


def _prop_body(xr_ref, nz_ref, obs_ref, ab_ref, t_ref, xn_ref, lnw_ref):
    x = xr_ref[0]
    z = jnp.dot(x, ab_ref[...], preferred_element_type=jnp.float32)
    xn = z + 0.1 * nz_ref[0]
    xn_ref[0] = xn
    dfv = xn - obs_ref[0]
    sq = dfv * dfv
    q = jnp.dot(sq, t_ref[...], preferred_element_type=jnp.float32)
    g = -0.5 * q
    m = jnp.max(g)
    lse = jnp.log(jnp.sum(jnp.exp(g - m))) + m
    lnw_ref[0] = g - lse


def _propagate(xr_v, nz_v, obs_t, a_big, t_sel):
    return pl.pallas_call(
        _prop_body,
        grid=(_B,),
        in_specs=[
            pl.BlockSpec((1, _N // 8, 128), lambda b: (b, 0, 0)),
            pl.BlockSpec((1, _N // 8, 128), lambda b: (b, 0, 0)),
            pl.BlockSpec((1, 1, 128), lambda b: (b, 0, 0)),
            pl.BlockSpec((128, 128), lambda b: (0, 0)),
            pl.BlockSpec((128, 8), lambda b: (0, 0)),
        ],
        out_specs=[
            pl.BlockSpec((1, _N // 8, 128), lambda b: (b, 0, 0)),
            pl.BlockSpec((1, _N // 8, 8), lambda b: (b, 0, 0)),
        ],
        out_shape=[
            jax.ShapeDtypeStruct((_B, _N // 8, 128), jnp.float32),
            jax.ShapeDtypeStruct((_B, _N // 8, 8), jnp.float32),
        ],
    )(xr_v, nz_v, obs_t, a_big, t_sel)


def kernel(x_t, log_weights, obs, noise, u0, A):
    lnw = log_weights - jax.scipy.special.logsumexp(
        log_weights, axis=-1, keepdims=True)
    w = jnp.exp(lnw)
    cdf = jnp.cumsum(w, axis=1)
    cdf_norm = cdf / cdf[:, -1:]

    s = _compute_s(cdf_norm, u0)

    x_res = _resample_gather(s, x_t.reshape(_B * _N, _D))

    a_big = jnp.kron(jnp.eye(8, dtype=jnp.float32), A)
    t_sel = jnp.kron(jnp.eye(8, dtype=jnp.float32),
                     jnp.ones((16, 1), jnp.float32))
    obs_t = jnp.tile(obs, (1, 8)).reshape(_B, 1, 128)
    xn, lnw_new = _propagate(
        x_res.reshape(_B, _N // 8, 128),
        noise.reshape(_B, _N // 8, 128),
        obs_t, a_big, t_sel)

    return jnp.concatenate(
        [xn.reshape(_B, _N, _D), lnw_new.reshape(_B, _N, 1)], axis=-1)


# speedup vs baseline: 42.8077x; 1.0198x over previous
import functools

import jax
import jax.numpy as jnp
from jax import lax
from jax.experimental import pallas as pl
from jax.experimental.pallas import tpu as pltpu
from jax.experimental.pallas import tpu_sc as plsc

_B, _N, _D = 32, 65536, 16
_RPW = 512
_CH_S = 8192
_G_ROWS = 2048
_H_PAD = _N + 128


def _lane_shift(x, k):
    rolled = pltpu.roll(x, k, axis=1)
    lane = lax.broadcasted_iota(jnp.int32, x.shape, 1)
    return jnp.where(lane >= k, rolled, 0.0)


def _s_body(w_ref, u0_ref, s_ref, et_ref, s2_ref):
    e = w_ref[0]
    et_ref[...] = jnp.swapaxes(e, 0, 1)

    def step(k, c):
        et_ref[pl.ds(k + 1, 1), :] = (
            et_ref[pl.ds(k + 1, 1), :] + et_ref[pl.ds(k, 1), :])
        return c

    lax.fori_loop(0, 127, step, 0)
    rs = et_ref[127:128, :]
    for g in range(4):
        s2_ref[:, g:g + 1] = jnp.swapaxes(rs[:, 128 * g:128 * (g + 1)], 0, 1)

    def step2(k, c):
        s2_ref[pl.ds(k + 1, 1), :] = (
            s2_ref[pl.ds(k + 1, 1), :] + s2_ref[pl.ds(k, 1), :])
        return c

    lax.fori_loop(0, 127, step2, 0)
    t = s2_ref[127:128, :]
    exc = (_lane_shift(t, 3) + _lane_shift(t, 2)) + _lane_shift(t, 1)
    incl = s2_ref[:, 0:4] + exc[:, 0:4]
    t4 = jnp.swapaxes(incl, 0, 1)
    rowoff_incl = jnp.concatenate(
        [t4[g:g + 1, :] for g in range(4)], axis=1)
    rowoff_excl = _lane_shift(rowoff_incl, 1)
    cdf_t = et_ref[...] + rowoff_excl
    total = cdf_t[127:128, 511:512]
    v_t = (cdf_t / total) * 65536.0
    v = jnp.swapaxes(v_t, 0, 1)

    u0 = u0_ref[0, 0, 0]
    gf = jnp.floor(v - u0)
    gc = jnp.clip(gf, 0.0, 65535.0)
    acc = jnp.maximum(gc - 2.0, 0.0)
    for d in range(-2, 3):
        jp = gc + float(d)
        ok = (jp >= 0.0) & (jp <= 65535.0) & ((jp + u0) <= v)
        acc = acc + jnp.where(ok, 1.0, 0.0)
    s_ref[0] = acc.astype(jnp.int32)


def _compute_s(w, u0):
    w_v = w.reshape(_B, _RPW, 128)
    u0_v = u0.reshape(_B, 1, 1)
    return pl.pallas_call(
        _s_body,
        grid=(_B,),
        in_specs=[
            pl.BlockSpec((1, _RPW, 128), lambda b: (b, 0, 0)),
            pl.BlockSpec((1, 1, 1), lambda b: (b, 0, 0)),
        ],
        out_specs=pl.BlockSpec((1, _RPW, 128), lambda b: (b, 0, 0)),
        out_shape=jax.ShapeDtypeStruct((_B, _RPW, 128), jnp.int32),
        scratch_shapes=[
            pltpu.VMEM((128, _RPW), jnp.float32),
            pltpu.VMEM((128, 128), jnp.float32),
        ],
    )(w_v, u0_v)


def _sc_resample(s_hbm, x_hbm, z_hbm, out_hbm, h, sbuf, gbuf, sem):
    wid = lax.axis_index("s") * 2 + lax.axis_index("c")
    base = wid * _N
    ones16 = jnp.ones((16,), jnp.int32)

    pltpu.sync_copy(z_hbm, h)

    def chunk_body(c, carry):
        pltpu.sync_copy(s_hbm.at[pl.ds(base + c * _CH_S, _CH_S)], sbuf)

        def scat(i, cc):
            sv = sbuf[pl.ds(i * 16, 16)]
            plsc.addupdate_scatter(h, [sv], ones16)
            return cc

        lax.fori_loop(0, _CH_S // 16, scat, 0)
        return carry

    lax.fori_loop(0, _N // _CH_S, chunk_body, 0)

    def scan_body(i, carry):
        v = h[pl.ds(i * 16, 16)]
        cs = plsc.cumsum(v)
        h[pl.ds(i * 16, 16)] = cs + (carry + base)
        return carry + lax.reduce_max(cs, (0,))

    lax.fori_loop(0, _N // 16, scan_body, 0)

    def gout(t, carry):
        cps = []
        for m in range(_G_ROWS // 128):
            idxs = h.at[pl.ds(t * _G_ROWS + m * 128, 128)]
            cps.append(pltpu.async_copy(
                x_hbm.at[idxs], gbuf.at[pl.ds(m * 128, 128), :], sem))
        for cp in cps:
            cp.wait()
        pltpu.sync_copy(
            gbuf, out_hbm.at[pl.ds(base + t * _G_ROWS, _G_ROWS), :])
        return carry

    lax.fori_loop(0, _N // _G_ROWS, gout, 0)


def _resample_gather(s, x_flat):
    mesh = plsc.VectorSubcoreMesh(core_axis_name="c", subcore_axis_name="s")
    kern = functools.partial(
        pl.kernel,
        mesh=mesh,
        out_type=jax.ShapeDtypeStruct((_B * _N, _D), jnp.float32),
        scratch_types=[
            pltpu.VMEM((_H_PAD,), jnp.int32),
            pltpu.VMEM((_CH_S,), jnp.int32),
            pltpu.VMEM((_G_ROWS, _D), jnp.float32),
            pltpu.SemaphoreType.DMA,
        ],
        compiler_params=pltpu.CompilerParams(
            needs_layout_passes=False, use_tc_tiling_on_sc=False),
    )(_sc_resample)
    zeros_h = jnp.zeros((_H_PAD,), jnp.int32)
    return kern(s.reshape(_B * _N), x_flat, zeros_h)


def _prop_body(xr_ref, nz_ref, obs_ref, ab_ref, t_ref, xn_ref, lnw_ref):
    x = xr_ref[0]
    z = jnp.dot(x, ab_ref[...], preferred_element_type=jnp.float32)
    xn = z + 0.1 * nz_ref[0]
    xn_ref[0] = xn
    dfv = xn - obs_ref[0]
    sq = dfv * dfv
    q = jnp.dot(sq, t_ref[...], preferred_element_type=jnp.float32)
    g = -0.5 * q
    m = jnp.max(g)
    lse = jnp.log(jnp.sum(jnp.exp(g - m))) + m
    lnw_ref[0] = g - lse


def _propagate(xr_v, nz_v, obs_t, a_big, t_sel):
    return pl.pallas_call(
        _prop_body,
        grid=(_B,),
        in_specs=[
            pl.BlockSpec((1, _N // 8, 128), lambda b: (b, 0, 0)),
            pl.BlockSpec((1, _N // 8, 128), lambda b: (b, 0, 0)),
            pl.BlockSpec((1, 1, 128), lambda b: (b, 0, 0)),
            pl.BlockSpec((128, 128), lambda b: (0, 0)),
            pl.BlockSpec((128, 8), lambda b: (0, 0)),
        ],
        out_specs=[
            pl.BlockSpec((1, _N // 8, 128), lambda b: (b, 0, 0)),
            pl.BlockSpec((1, _N // 8, 8), lambda b: (b, 0, 0)),
        ],
        out_shape=[
            jax.ShapeDtypeStruct((_B, _N // 8, 128), jnp.float32),
            jax.ShapeDtypeStruct((_B, _N // 8, 8), jnp.float32),
        ],
    )(xr_v, nz_v, obs_t, a_big, t_sel)


def kernel(x_t, log_weights, obs, noise, u0, A):
    lnw = log_weights - jax.scipy.special.logsumexp(
        log_weights, axis=-1, keepdims=True)
    w = jnp.exp(lnw)

    s = _compute_s(w, u0)

    x_res = _resample_gather(s, x_t.reshape(_B * _N, _D))

    a_big = jnp.kron(jnp.eye(8, dtype=jnp.float32), A)
    t_sel = jnp.kron(jnp.eye(8, dtype=jnp.float32),
                     jnp.ones((16, 1), jnp.float32))
    obs_t = jnp.tile(obs, (1, 8)).reshape(_B, 1, 128)
    xn, lnw_new = _propagate(
        x_res.reshape(_B, _N // 8, 128),
        noise.reshape(_B, _N // 8, 128),
        obs_t, a_big, t_sel)

    return jnp.concatenate(
        [xn.reshape(_B, _N, _D), lnw_new.reshape(_B, _N, 1)], axis=-1)
